# baseline (speedup 1.0000x reference)
import jax
import jax.numpy as jnp
from jax.experimental import pallas as pl


def _mm(a, b, dims):
    return jax.lax.dot_general(a, b, (dims, ((), ())),
                               preferred_element_type=jnp.float32)


def _dyn_graph_wave_kernel(n1_ref, xp_ref, wself_ref, wagg_ref, out_ref):
    N = n1_ref.shape[0]
    NT = 4
    T = N // NT
    BF = xp_ref.shape[1]
    F = wself_ref.shape[0]
    B = BF // F

    def masked(logits):
        s = jax.nn.sigmoid(logits)
        return jnp.where(s > 0.5, s, 0.0)

    n1t = [n1_ref[i * T:(i + 1) * T, :] for i in range(NT)]
    xpt = [xp_ref[i * T:(i + 1) * T, :] for i in range(NT)]
    w = {}
    for i in range(NT):
        for j in range(i, NT):
            w[(i, j)] = masked(_mm(n1t[i], n1t[j], ((1,), (1,))))
    aggs = []
    for i in range(NT):
        acc = None
        for j in range(NT):
            if j <= i:
                term = _mm(w[(j, i)], xpt[j], ((0,), (0,)))
            else:
                term = _mm(w[(i, j)], xpt[j], ((1,), (0,)))
            acc = term if acc is None else acc + term
        aggs.append(acc)
    agg = jnp.concatenate(aggs, axis=0)
    bi = jax.lax.broadcasted_iota(jnp.int32, (BF, BF), 0) // F
    bj = jax.lax.broadcasted_iota(jnp.int32, (BF, BF), 1) // F
    blk = (bi == bj).astype(jnp.float32)
    wagg_blk = jnp.tile(wagg_ref[...], (B, B)) * blk
    wself_blk = jnp.tile(wself_ref[...], (B, B)) * blk
    out_ref[...] = (
        _mm(agg, wagg_blk, ((1,), (0,)))
        + _mm(xp_ref[...], wself_blk, ((1,), (0,)))
    )


def kernel(x, ptr, node1, W_self, W_agg):
    del ptr
    N, _ = node1.shape
    n, F = x.shape
    B = n // N
    xp = x.reshape(B, N, F).transpose(1, 0, 2).reshape(N, B * F)
    out_p = pl.pallas_call(
        _dyn_graph_wave_kernel,
        out_shape=jax.ShapeDtypeStruct((N, B * F), x.dtype),
    )(node1, xp, W_self, W_agg)
    return out_p.reshape(N, B, F).transpose(1, 0, 2).reshape(n, F)
